# baseline (speedup 1.0000x reference)
import functools

import jax
import jax.numpy as jnp
from jax import lax
from jax.experimental import pallas as pl
from jax.experimental.pallas import tpu as pltpu

N = 10000
E = 160000
D_IN = 256
H = 256
HEADS = 4
C = H // HEADS
L = 4

ROW_BLK = 400


def _gelu(v):
    return 0.5 * v * (1.0 + lax.erf(v * 0.7071067811865476))


def _proj_body(x_ref, wp_ref, bp_ref, out_ref):
    out_ref[...] = jnp.dot(x_ref[...], wp_ref[...],
                           preferred_element_type=jnp.float32) + bp_ref[...]


def tc_proj(x, Wp, bp):
    grid = (N // ROW_BLK,)
    return pl.pallas_call(
        _proj_body,
        grid=grid,
        in_specs=[
            pl.BlockSpec((ROW_BLK, D_IN), lambda i: (i, 0)),
            pl.BlockSpec((D_IN, H), lambda i: (0, 0)),
            pl.BlockSpec((1, H), lambda i: (0, 0)),
        ],
        out_specs=pl.BlockSpec((ROW_BLK, H), lambda i: (i, 0)),
        out_shape=jax.ShapeDtypeStruct((N, H), jnp.float32),
    )(x, Wp, bp.reshape(1, H))


def _mm2_body(h_ref, wl_ref, wr_ref, xl_ref, xr_ref):
    h = h_ref[...]
    xl_ref[...] = jnp.dot(h, wl_ref[...], preferred_element_type=jnp.float32)
    xr_ref[...] = jnp.dot(h, wr_ref[...], preferred_element_type=jnp.float32)


def tc_mm2(h, Wl_i, Wr_i):
    grid = (N // ROW_BLK,)
    return pl.pallas_call(
        _mm2_body,
        grid=grid,
        in_specs=[
            pl.BlockSpec((ROW_BLK, H), lambda i: (i, 0)),
            pl.BlockSpec((H, H), lambda i: (0, 0)),
            pl.BlockSpec((H, H), lambda i: (0, 0)),
        ],
        out_specs=[
            pl.BlockSpec((ROW_BLK, H), lambda i: (i, 0)),
            pl.BlockSpec((ROW_BLK, H), lambda i: (i, 0)),
        ],
        out_shape=[
            jax.ShapeDtypeStruct((N, H), jnp.float32),
            jax.ShapeDtypeStruct((N, H), jnp.float32),
        ],
    )(h, Wl_i, Wr_i)


def _layer_body(agg_ref, hprev_ref, bc_ref, wl_ref, wr_ref,
                h_ref, xl_ref, xr_ref):
    h = _gelu(agg_ref[...] + bc_ref[...]) + hprev_ref[...]
    h_ref[...] = h
    xl_ref[...] = jnp.dot(h, wl_ref[...], preferred_element_type=jnp.float32)
    xr_ref[...] = jnp.dot(h, wr_ref[...], preferred_element_type=jnp.float32)


def tc_layer(agg, h_prev, bc_i, Wl_i, Wr_i):
    grid = (N // ROW_BLK,)
    return pl.pallas_call(
        _layer_body,
        grid=grid,
        in_specs=[
            pl.BlockSpec((ROW_BLK, H), lambda i: (i, 0)),
            pl.BlockSpec((ROW_BLK, H), lambda i: (i, 0)),
            pl.BlockSpec((1, H), lambda i: (0, 0)),
            pl.BlockSpec((H, H), lambda i: (0, 0)),
            pl.BlockSpec((H, H), lambda i: (0, 0)),
        ],
        out_specs=[
            pl.BlockSpec((ROW_BLK, H), lambda i: (i, 0)),
            pl.BlockSpec((ROW_BLK, H), lambda i: (i, 0)),
            pl.BlockSpec((ROW_BLK, H), lambda i: (i, 0)),
        ],
        out_shape=[
            jax.ShapeDtypeStruct((N, H), jnp.float32),
            jax.ShapeDtypeStruct((N, H), jnp.float32),
            jax.ShapeDtypeStruct((N, H), jnp.float32),
        ],
    )(agg, h_prev, bc_i.reshape(1, H), Wl_i, Wr_i)


def _head_body(agg_ref, hprev_ref, bc_ref, gamma_ref, beta_ref,
               w1_ref, b1_ref, w2_ref, b2_ref, y_ref):
    h = _gelu(agg_ref[...] + bc_ref[...]) + hprev_ref[...]
    mu = jnp.mean(h, axis=-1, keepdims=True)
    var = jnp.mean((h - mu) ** 2, axis=-1, keepdims=True)
    hn = (h - mu) / jnp.sqrt(var + 1e-5) * gamma_ref[...] + beta_ref[...]
    z = _gelu(jnp.dot(hn, w1_ref[...], preferred_element_type=jnp.float32)
              + b1_ref[...])
    y_ref[...] = jnp.sum(z * w2_ref[...], axis=-1, keepdims=True) + b2_ref[...]


def tc_head(agg, h_prev, bc_i, gamma, beta, W1, b1, W2, b2):
    grid = (N // ROW_BLK,)
    return pl.pallas_call(
        _head_body,
        grid=grid,
        in_specs=[
            pl.BlockSpec((ROW_BLK, H), lambda i: (i, 0)),
            pl.BlockSpec((ROW_BLK, H), lambda i: (i, 0)),
            pl.BlockSpec((1, H), lambda i: (0, 0)),
            pl.BlockSpec((1, H), lambda i: (0, 0)),
            pl.BlockSpec((1, H), lambda i: (0, 0)),
            pl.BlockSpec((H, H), lambda i: (0, 0)),
            pl.BlockSpec((1, H), lambda i: (0, 0)),
            pl.BlockSpec((1, H), lambda i: (0, 0)),
            pl.BlockSpec((1, 1), lambda i: (0, 0)),
        ],
        out_specs=pl.BlockSpec((ROW_BLK, 1), lambda i: (i, 0)),
        out_shape=jax.ShapeDtypeStruct((N, 1), jnp.float32),
    )(agg, h_prev, bc_i.reshape(1, H), gamma.reshape(1, H),
      beta.reshape(1, H), W1, b1.reshape(1, H), W2.reshape(1, H),
      b2.reshape(1, 1))


def edge_phase(xl, xr, att_i, src, dst):
    e = jax.nn.leaky_relu(
        xl.reshape(N, HEADS, C)[src] + xr.reshape(N, HEADS, C)[dst],
        negative_slope=0.2)
    logit = (e * att_i[None, :, :]).sum(-1)
    m = jax.ops.segment_max(logit, dst, num_segments=N)
    a = jnp.exp(logit - m[dst])
    den = jax.ops.segment_sum(a, dst, num_segments=N)
    a = a / (den[dst] + 1e-16)
    out = jax.ops.segment_sum(
        a[:, :, None] * xl.reshape(N, HEADS, C)[src], dst, num_segments=N)
    return out.reshape(N, H)


def kernel(x, edge_index, Wp, bp, Wl, Wr, att, bc, gamma, beta,
           W1, b1, W2, b2):
    loop = jnp.arange(N, dtype=edge_index.dtype)
    src = jnp.concatenate([edge_index[0], loop])
    dst = jnp.concatenate([edge_index[1], loop])

    h = tc_proj(x, Wp, bp)
    xl, xr = tc_mm2(h, Wl[0], Wr[0])
    agg = edge_phase(xl, xr, att[0], src, dst)
    for i in range(1, L):
        h, xl, xr = tc_layer(agg, h, bc[i - 1], Wl[i], Wr[i])
        agg = edge_phase(xl, xr, att[i], src, dst)
    y = tc_head(agg, h, bc[L - 1], gamma, beta, W1, b1, W2, b2)
    return y


# speedup vs baseline: 2.4771x; 2.4771x over previous
import functools

import jax
import jax.numpy as jnp
from jax import lax
from jax.experimental import pallas as pl
from jax.experimental.pallas import tpu as pltpu
from jax.experimental.pallas import tpu_sc as plsc

N = 10000
E = 160000
D_IN = 256
H = 256
HEADS = 4
C = H // HEADS
L = 4

ROW_BLK = 400


def _gelu(v):
    return 0.5 * v * (1.0 + lax.erf(v * 0.7071067811865476))


def _proj_body(x_ref, wp_ref, bp_ref, out_ref):
    out_ref[...] = jnp.dot(x_ref[...], wp_ref[...],
                           preferred_element_type=jnp.float32) + bp_ref[...]


def tc_proj(x, Wp, bp):
    grid = (N // ROW_BLK,)
    return pl.pallas_call(
        _proj_body,
        grid=grid,
        in_specs=[
            pl.BlockSpec((ROW_BLK, D_IN), lambda i: (i, 0)),
            pl.BlockSpec((D_IN, H), lambda i: (0, 0)),
            pl.BlockSpec((1, H), lambda i: (0, 0)),
        ],
        out_specs=pl.BlockSpec((ROW_BLK, H), lambda i: (i, 0)),
        out_shape=jax.ShapeDtypeStruct((N, H), jnp.float32),
    )(x, Wp, bp.reshape(1, H))


def _mm2_body(h_ref, wl_ref, wr_ref, xl_ref, xr_ref):
    h = h_ref[...]
    xl_ref[...] = jnp.dot(h, wl_ref[...], preferred_element_type=jnp.float32)
    xr_ref[...] = jnp.dot(h, wr_ref[...], preferred_element_type=jnp.float32)


def tc_mm2(h, Wl_i, Wr_i):
    grid = (N // ROW_BLK,)
    return pl.pallas_call(
        _mm2_body,
        grid=grid,
        in_specs=[
            pl.BlockSpec((ROW_BLK, H), lambda i: (i, 0)),
            pl.BlockSpec((H, H), lambda i: (0, 0)),
            pl.BlockSpec((H, H), lambda i: (0, 0)),
        ],
        out_specs=[
            pl.BlockSpec((ROW_BLK, H), lambda i: (i, 0)),
            pl.BlockSpec((ROW_BLK, H), lambda i: (i, 0)),
        ],
        out_shape=[
            jax.ShapeDtypeStruct((N, H), jnp.float32),
            jax.ShapeDtypeStruct((N, H), jnp.float32),
        ],
    )(h, Wl_i, Wr_i)


def _layer_body(agg_ref, hprev_ref, bc_ref, wl_ref, wr_ref,
                h_ref, xl_ref, xr_ref):
    h = _gelu(agg_ref[...] + bc_ref[...]) + hprev_ref[...]
    h_ref[...] = h
    xl_ref[...] = jnp.dot(h, wl_ref[...], preferred_element_type=jnp.float32)
    xr_ref[...] = jnp.dot(h, wr_ref[...], preferred_element_type=jnp.float32)


def tc_layer(agg, h_prev, bc_i, Wl_i, Wr_i):
    grid = (N // ROW_BLK,)
    return pl.pallas_call(
        _layer_body,
        grid=grid,
        in_specs=[
            pl.BlockSpec((ROW_BLK, H), lambda i: (i, 0)),
            pl.BlockSpec((ROW_BLK, H), lambda i: (i, 0)),
            pl.BlockSpec((1, H), lambda i: (0, 0)),
            pl.BlockSpec((H, H), lambda i: (0, 0)),
            pl.BlockSpec((H, H), lambda i: (0, 0)),
        ],
        out_specs=[
            pl.BlockSpec((ROW_BLK, H), lambda i: (i, 0)),
            pl.BlockSpec((ROW_BLK, H), lambda i: (i, 0)),
            pl.BlockSpec((ROW_BLK, H), lambda i: (i, 0)),
        ],
        out_shape=[
            jax.ShapeDtypeStruct((N, H), jnp.float32),
            jax.ShapeDtypeStruct((N, H), jnp.float32),
            jax.ShapeDtypeStruct((N, H), jnp.float32),
        ],
    )(agg, h_prev, bc_i.reshape(1, H), Wl_i, Wr_i)


def _head_body(agg_ref, hprev_ref, bc_ref, gamma_ref, beta_ref,
               w1_ref, b1_ref, w2_ref, b2_ref, y_ref):
    h = _gelu(agg_ref[...] + bc_ref[...]) + hprev_ref[...]
    mu = jnp.mean(h, axis=-1, keepdims=True)
    var = jnp.mean((h - mu) ** 2, axis=-1, keepdims=True)
    hn = (h - mu) / jnp.sqrt(var + 1e-5) * gamma_ref[...] + beta_ref[...]
    z = _gelu(jnp.dot(hn, w1_ref[...], preferred_element_type=jnp.float32)
              + b1_ref[...])
    y_ref[...] = jnp.sum(z * w2_ref[...], axis=-1, keepdims=True) + b2_ref[...]


def tc_head(agg, h_prev, bc_i, gamma, beta, W1, b1, W2, b2):
    grid = (N // ROW_BLK,)
    return pl.pallas_call(
        _head_body,
        grid=grid,
        in_specs=[
            pl.BlockSpec((ROW_BLK, H), lambda i: (i, 0)),
            pl.BlockSpec((ROW_BLK, H), lambda i: (i, 0)),
            pl.BlockSpec((1, H), lambda i: (0, 0)),
            pl.BlockSpec((1, H), lambda i: (0, 0)),
            pl.BlockSpec((1, H), lambda i: (0, 0)),
            pl.BlockSpec((H, H), lambda i: (0, 0)),
            pl.BlockSpec((1, H), lambda i: (0, 0)),
            pl.BlockSpec((1, H), lambda i: (0, 0)),
            pl.BlockSpec((1, 1), lambda i: (0, 0)),
        ],
        out_specs=pl.BlockSpec((ROW_BLK, 1), lambda i: (i, 0)),
        out_shape=jax.ShapeDtypeStruct((N, 1), jnp.float32),
    )(agg, h_prev, bc_i.reshape(1, H), gamma.reshape(1, H),
      beta.reshape(1, H), W1, b1.reshape(1, H), W2.reshape(1, H),
      b2.reshape(1, 1))


E_SELF = E + N
CHUNK = 128
E_PAD = 172032
A_PER_TILE = E_PAD // 32
NPAD4 = 40960
NSEG = NPAD4 // 16
NPT = 320
ACCR = 328
TRASH_ROW = 324
ASUP = 2048
ACH = 64

NEG_BIG = -1e30

_CP = pltpu.CompilerParams(needs_layout_passes=False)


def _iota16():
    return lax.iota(jnp.int32, 16)


def _fill(ref, start, nvec, value):
    def body(k, _):
        ref[pl.ds(start + 16 * k, 16)] = jnp.full((16,), value, ref.dtype)
        return 0
    lax.fori_loop(0, nvec, body, 0)


def _sc_mesh():
    return plsc.VectorSubcoreMesh(core_axis_name="c", subcore_axis_name="s")


def make_sc_logits():
    mesh = _sc_mesh()

    @functools.partial(
        pl.kernel,
        out_type=jax.ShapeDtypeStruct((E_PAD * HEADS,), jnp.float32),
        mesh=mesh,
        compiler_params=_CP,
        scratch_types=[
            pltpu.VMEM((A_PER_TILE,), jnp.int32),
            pltpu.VMEM((A_PER_TILE,), jnp.int32),
            pltpu.VMEM((CHUNK, H), jnp.float32),
            pltpu.VMEM((CHUNK, H), jnp.float32),
            pltpu.VMEM((A_PER_TILE * HEADS,), jnp.float32),
            pltpu.VMEM((H,), jnp.float32),
            pltpu.SemaphoreType.DMA,
            pltpu.SemaphoreType.DMA,
        ],
    )
    def sc_logits(src_hbm, dst_hbm, xl_hbm, xr_hbm, att_hbm, logit_hbm,
                  srca_v, dsta_v, rows_l, rows_r, lga_v, att_v, sem1, sem2):
        p = lax.axis_index("c")
        s = lax.axis_index("s")
        w = p * 16 + s
        base_w = w * A_PER_TILE

        pltpu.sync_copy(att_hbm, att_v)
        pltpu.sync_copy(src_hbm.at[pl.ds(base_w, A_PER_TILE)], srca_v)
        pltpu.sync_copy(dst_hbm.at[pl.ds(base_w, A_PER_TILE)], dsta_v)

        def clamp_body(k, _):
            v = dsta_v[pl.ds(16 * k, 16)]
            dsta_v[pl.ds(16 * k, 16)] = jnp.minimum(v, N - 1)
            return 0
        lax.fori_loop(0, A_PER_TILE // 16, clamp_body, 0)
        _fill(lga_v, 0, A_PER_TILE * HEADS // 16, 0.0)

        def chunk_body(c, _):
            cb = c * CHUNK
            cl = pltpu.async_copy(
                xl_hbm.at[srca_v.at[pl.ds(cb, CHUNK)]], rows_l, sem1)
            cr = pltpu.async_copy(
                xr_hbm.at[dsta_v.at[pl.ds(cb, CHUNK)]], rows_r, sem2)
            cl.wait()
            cr.wait()

            def edge_body(j, _):
                lbase = (cb + j) * HEADS
                for k in range(16):
                    t = (rows_l[j, pl.ds(16 * k, 16)]
                         + rows_r[j, pl.ds(16 * k, 16)])
                    lr = jnp.maximum(t, 0.2 * t)
                    prod = lr * att_v[pl.ds(16 * k, 16)]
                    idx = jnp.full((16,), lbase + (k // 4), jnp.int32)
                    plsc.addupdate_scatter(lga_v, [idx], prod)
                return 0
            lax.fori_loop(0, CHUNK, edge_body, 0)
            return 0
        lax.fori_loop(0, A_PER_TILE // CHUNK, chunk_body, 0)

        pltpu.sync_copy(
            lga_v, logit_hbm.at[pl.ds(base_w * HEADS, A_PER_TILE * HEADS)])

    return sc_logits


def make_sc_stats():
    mesh = _sc_mesh()

    @functools.partial(
        pl.kernel,
        out_type=[jax.ShapeDtypeStruct((2, NPAD4), jnp.float32),
                  jax.ShapeDtypeStruct((2, NPAD4), jnp.float32),
                  jax.ShapeDtypeStruct((32, NPAD4), jnp.float32)],
        mesh=mesh,
        compiler_params=_CP,
        scratch_types=[
            pltpu.VMEM((NPAD4,), jnp.float32),
            pltpu.VMEM((NPAD4,), jnp.float32),
            pltpu.VMEM((A_PER_TILE,), jnp.int32),
            pltpu.VMEM((A_PER_TILE * HEADS,), jnp.float32),
            pltpu.VMEM((NSEG,), jnp.float32),
            pltpu.VMEM((NSEG,), jnp.float32),
            pltpu.VMEM_SHARED((NPAD4,), jnp.float32),
        ],
    )
    def sc_stats(dst_hbm, logit_hbm, m_part, den_part, exch,
                 m_buf, den_buf, dsts_v, lgs_v, acc_v, tmp_v, final_t):
        c = lax.axis_index("c")
        s = lax.axis_index("s")
        w = c * 16 + s
        base_w = w * A_PER_TILE
        iota = _iota16()
        head = iota & 3
        quad = iota >> 2

        pltpu.sync_copy(dst_hbm.at[pl.ds(base_w, A_PER_TILE)], dsts_v)
        pltpu.sync_copy(
            logit_hbm.at[pl.ds(base_w * HEADS, A_PER_TILE * HEADS)], lgs_v)
        _fill(m_buf, 0, NPAD4 // 16, NEG_BIG)
        _fill(den_buf, 0, NPAD4 // 16, 0.0)

        def lane_fidx(v):
            dvec = plsc.load_gather(dsts_v, [4 * v + quad])
            return dvec * 4 + head

        def max_body(v, _):
            fidx = lane_fidx(v)
            val = lgs_v[pl.ds(16 * v, 16)]
            cur = plsc.load_gather(m_buf, [fidx])

            def rbody(t, pn):
                plsc.store_scatter(m_buf, [fidx], val, mask=pn)
                cur2 = plsc.load_gather(m_buf, [fidx])
                return pn & (val > cur2)
            lax.fori_loop(0, 16, rbody, val > cur)
            return 0
        lax.fori_loop(0, A_PER_TILE * HEADS // 16, max_body, 0)

        def reduce_tables(buf_v, out_hbm, is_max):
            plsc.subcore_barrier()
            pltpu.sync_copy(buf_v, exch.at[w])
            plsc.subcore_barrier()
            seg0 = s * NSEG
            pltpu.sync_copy(exch.at[c * 16, pl.ds(seg0, NSEG)], acc_v)

            def tile_body(t, _):
                pltpu.sync_copy(exch.at[c * 16 + t, pl.ds(seg0, NSEG)], tmp_v)

                def vec_body(k, _):
                    a = acc_v[pl.ds(16 * k, 16)]
                    b = tmp_v[pl.ds(16 * k, 16)]
                    acc_v[pl.ds(16 * k, 16)] = (
                        jnp.maximum(a, b) if is_max else a + b)
                    return 0
                lax.fori_loop(0, NSEG // 16, vec_body, 0)
                return 0
            lax.fori_loop(1, 16, tile_body, 0)
            pltpu.sync_copy(acc_v, out_hbm.at[c, pl.ds(seg0, NSEG)])
            pltpu.sync_copy(acc_v, final_t.at[pl.ds(seg0, NSEG)])
            plsc.subcore_barrier()
            pltpu.sync_copy(final_t, buf_v)

        reduce_tables(m_buf, m_part, True)

        def den_body(v, _):
            fidx = lane_fidx(v)
            val = lgs_v[pl.ds(16 * v, 16)]
            mg = plsc.load_gather(m_buf, [fidx])
            a = jnp.exp(val - mg)
            plsc.addupdate_scatter(den_buf, [fidx], a)
            return 0
        lax.fori_loop(0, A_PER_TILE * HEADS // 16, den_body, 0)

        reduce_tables(den_buf, den_part, False)

    return sc_stats


def make_sc_agg():
    mesh = _sc_mesh()

    @functools.partial(
        pl.kernel,
        out_type=jax.ShapeDtypeStruct((32 * NPT, H), jnp.float32),
        mesh=mesh,
        compiler_params=_CP,
        scratch_types=[
            pltpu.VMEM((ACCR, H), jnp.float32),
            pltpu.VMEM((ASUP + 16,), jnp.int32),
            pltpu.VMEM((ASUP + 16,), jnp.int32),
            pltpu.VMEM((ASUP * HEADS + 16,), jnp.float32),
            pltpu.VMEM((ASUP + 64,), jnp.int32),
            pltpu.VMEM((ACH, H), jnp.float32),
            pltpu.VMEM((ACH,), jnp.int32),
            pltpu.VMEM((ACH + 16,), jnp.int32),
            pltpu.VMEM((ACH * HEADS + 16,), jnp.float32),
            pltpu.VMEM((NPT * HEADS,), jnp.float32),
            pltpu.VMEM((NPT * HEADS,), jnp.float32),
            pltpu.VMEM((NPT * HEADS,), jnp.float32),
            pltpu.VMEM((NPT * HEADS,), jnp.float32),
            pltpu.SemaphoreType.DMA,
        ],
    )
    def sc_agg(src_hbm, dst_hbm, logit_hbm, m_part, den_part, xl_hbm,
               agg_hbm, acc, srcc_v, dstc_v, lgc_v, eids_v, rows_v, idx_v,
               drow_v, a_ch, m_loc, rd_loc, tpa, tpb, sem):
        c = lax.axis_index("c")
        s = lax.axis_index("s")
        w = c * 16 + s
        node0 = w * NPT
        iota = _iota16()
        head = iota & 3
        quad = iota >> 2

        t0 = node0 * HEADS
        pltpu.sync_copy(m_part.at[0, pl.ds(t0, NPT * HEADS)], m_loc)
        pltpu.sync_copy(m_part.at[1, pl.ds(t0, NPT * HEADS)], tpa)
        pltpu.sync_copy(den_part.at[0, pl.ds(t0, NPT * HEADS)], rd_loc)
        pltpu.sync_copy(den_part.at[1, pl.ds(t0, NPT * HEADS)], tpb)

        def merge_body(k, _):
            sl = pl.ds(16 * k, 16)
            m0 = m_loc[sl]
            m1 = tpa[sl]
            d0 = rd_loc[sl]
            d1 = tpb[sl]
            mm = jnp.maximum(m0, m1)
            den = d0 * jnp.exp(m0 - mm) + d1 * jnp.exp(m1 - mm)
            m_loc[sl] = mm
            rd_loc[sl] = 1.0 / (den + 1e-16)
            return 0
        lax.fori_loop(0, NPT * HEADS // 16, merge_body, 0)

        def zrow(j, _):
            for k in range(16):
                acc[j, pl.ds(16 * k, 16)] = jnp.zeros((16,), jnp.float32)
            return 0
        lax.fori_loop(0, ACCR, zrow, 0)

        def super_body(sc, _):
            eb = sc * ASUP
            pltpu.sync_copy(src_hbm.at[pl.ds(eb, ASUP)],
                            srcc_v.at[pl.ds(0, ASUP)])
            pltpu.sync_copy(dst_hbm.at[pl.ds(eb, ASUP)],
                            dstc_v.at[pl.ds(0, ASUP)])
            pltpu.sync_copy(
                logit_hbm.at[pl.ds(eb * HEADS, ASUP * HEADS)],
                lgc_v.at[pl.ds(0, ASUP * HEADS)])
            srcc_v[pl.ds(ASUP, 16)] = jnp.zeros((16,), jnp.int32)
            dstc_v[pl.ds(ASUP, 16)] = jnp.full((16,), -1, jnp.int32)
            _fill(eids_v, 0, (ASUP + 64) // 16, ASUP)

            def cmp_body(v, fill):
                dv = dstc_v[pl.ds(16 * v, 16)]
                dloc = dv - node0
                match = (dloc >= 0) & (dloc < NPT)
                ev = iota + 16 * v
                plsc.store_compressed(eids_v.at[pl.ds(fill, 16)], ev,
                                      mask=match)
                pc = plsc.all_reduce_population_count(match)
                return fill + pc[0]
            fill = lax.fori_loop(0, ASUP // 16, cmp_body, 0)
            nch = (fill + ACH - 1) // ACH

            def chunk_body(j, _):
                cb = j * ACH

                def bld_body(v, _):
                    elv = eids_v[pl.ds(cb + 16 * v, 16)]
                    srcs = plsc.load_gather(srcc_v, [elv])
                    idx_v[pl.ds(16 * v, 16)] = srcs
                    dvv = plsc.load_gather(dstc_v, [elv])
                    dloc = dvv - node0
                    drow_v[pl.ds(16 * v, 16)] = jnp.where(
                        dloc >= 0, dloc, TRASH_ROW)
                    return 0
                lax.fori_loop(0, ACH // 16, bld_body, 0)

                def aw_body(v, _):
                    elq = plsc.load_gather(eids_v, [cb + 4 * v + quad])
                    lg = plsc.load_gather(lgc_v, [elq * 4 + head])
                    dvq = plsc.load_gather(dstc_v, [elq])
                    dloc = dvq - node0
                    ok = dloc >= 0
                    fidx = jnp.where(ok, dloc, 0) * 4 + head
                    mg = plsc.load_gather(m_loc, [fidx])
                    rd = plsc.load_gather(rd_loc, [fidx])
                    av = jnp.exp(lg - mg) * rd
                    a_ch[pl.ds(16 * v, 16)] = jnp.where(ok, av, 0.0)
                    return 0
                lax.fori_loop(0, ACH * HEADS // 16, aw_body, 0)

                pltpu.async_copy(xl_hbm.at[idx_v], rows_v, sem).wait()

                def acc_body(j2, _):
                    rv = drow_v[pl.ds(j2, 16)]
                    row = rv[0]
                    av4 = a_ch[pl.ds(HEADS * j2, 16)]
                    aa = (av4[0], av4[1], av4[2], av4[3])
                    for k in range(16):
                        sl = pl.ds(16 * k, 16)
                        acc[row, sl] = (acc[row, sl]
                                        + rows_v[j2, sl] * aa[k // 4])
                    return 0
                lax.fori_loop(0, ACH, acc_body, 0)
                return 0
            lax.fori_loop(0, nch, chunk_body, 0)
            return 0
        lax.fori_loop(0, E_PAD // ASUP, super_body, 0)

        pltpu.sync_copy(acc.at[pl.ds(0, NPT)], agg_hbm.at[pl.ds(node0, NPT)])

    return sc_agg


_SC_LOGITS = None
_SC_STATS = None
_SC_AGG = None


def _sc_kernels():
    global _SC_LOGITS, _SC_STATS, _SC_AGG
    if _SC_LOGITS is None:
        _SC_LOGITS = make_sc_logits()
        _SC_STATS = make_sc_stats()
        _SC_AGG = make_sc_agg()
    return _SC_LOGITS, _SC_STATS, _SC_AGG


def edge_phase(xl, xr, att_i, srcp, dstp):
    sc_logits, sc_stats, sc_agg = _sc_kernels()
    logit = sc_logits(srcp, dstp, xl, xr, att_i.reshape(H))
    m_part, den_part, _ = sc_stats(dstp, logit)
    agg_pad = sc_agg(srcp, dstp, logit, m_part, den_part, xl)
    return agg_pad[:N]


def kernel(x, edge_index, Wp, bp, Wl, Wr, att, bc, gamma, beta,
           W1, b1, W2, b2):
    loop = jnp.arange(N, dtype=jnp.int32)
    npad = E_PAD - E_SELF
    srcp = jnp.concatenate(
        [edge_index[0].astype(jnp.int32), loop,
         jnp.zeros((npad,), jnp.int32)])
    dstp = jnp.concatenate(
        [edge_index[1].astype(jnp.int32), loop,
         jnp.full((npad,), N, jnp.int32)])

    h = tc_proj(x, Wp, bp)
    xl, xr = tc_mm2(h, Wl[0], Wr[0])
    agg = edge_phase(xl, xr, att[0], srcp, dstp)
    for i in range(1, L):
        h, xl, xr = tc_layer(agg, h, bc[i - 1], Wl[i], Wr[i])
        agg = edge_phase(xl, xr, att[i], srcp, dstp)
    y = tc_head(agg, h, bc[L - 1], gamma, beta, W1, b1, W2, b2)
    return y


# speedup vs baseline: 3.3112x; 1.3367x over previous
import functools

import jax
import jax.numpy as jnp
from jax import lax
from jax.experimental import pallas as pl
from jax.experimental.pallas import tpu as pltpu
from jax.experimental.pallas import tpu_sc as plsc

N = 10000
E = 160000
D_IN = 256
H = 256
HEADS = 4
C = H // HEADS
L = 4

ROW_BLK = 400


def _gelu(v):
    return 0.5 * v * (1.0 + lax.erf(v * 0.7071067811865476))


def _proj_body(x_ref, wp_ref, bp_ref, out_ref):
    out_ref[...] = jnp.dot(x_ref[...], wp_ref[...],
                           preferred_element_type=jnp.float32) + bp_ref[...]


def tc_proj(x, Wp, bp):
    grid = (N // ROW_BLK,)
    return pl.pallas_call(
        _proj_body,
        grid=grid,
        in_specs=[
            pl.BlockSpec((ROW_BLK, D_IN), lambda i: (i, 0)),
            pl.BlockSpec((D_IN, H), lambda i: (0, 0)),
            pl.BlockSpec((1, H), lambda i: (0, 0)),
        ],
        out_specs=pl.BlockSpec((ROW_BLK, H), lambda i: (i, 0)),
        out_shape=jax.ShapeDtypeStruct((N, H), jnp.float32),
    )(x, Wp, bp.reshape(1, H))


def _mm2_body(h_ref, wl_ref, wr_ref, xl_ref, xr_ref):
    h = h_ref[...]
    xl_ref[...] = jnp.dot(h, wl_ref[...], preferred_element_type=jnp.float32)
    xr_ref[...] = jnp.dot(h, wr_ref[...], preferred_element_type=jnp.float32)


def tc_mm2(h, Wl_i, Wr_i):
    grid = (N // ROW_BLK,)
    return pl.pallas_call(
        _mm2_body,
        grid=grid,
        in_specs=[
            pl.BlockSpec((ROW_BLK, H), lambda i: (i, 0)),
            pl.BlockSpec((H, H), lambda i: (0, 0)),
            pl.BlockSpec((H, H), lambda i: (0, 0)),
        ],
        out_specs=[
            pl.BlockSpec((ROW_BLK, H), lambda i: (i, 0)),
            pl.BlockSpec((ROW_BLK, H), lambda i: (i, 0)),
        ],
        out_shape=[
            jax.ShapeDtypeStruct((N, H), jnp.float32),
            jax.ShapeDtypeStruct((N, H), jnp.float32),
        ],
    )(h, Wl_i, Wr_i)


def _layer_body(agg_ref, hprev_ref, bc_ref, wl_ref, wr_ref,
                h_ref, xl_ref, xr_ref):
    h = _gelu(agg_ref[...] + bc_ref[...]) + hprev_ref[...]
    h_ref[...] = h
    xl_ref[...] = jnp.dot(h, wl_ref[...], preferred_element_type=jnp.float32)
    xr_ref[...] = jnp.dot(h, wr_ref[...], preferred_element_type=jnp.float32)


def tc_layer(agg, h_prev, bc_i, Wl_i, Wr_i):
    grid = (N // ROW_BLK,)
    return pl.pallas_call(
        _layer_body,
        grid=grid,
        in_specs=[
            pl.BlockSpec((ROW_BLK, H), lambda i: (i, 0)),
            pl.BlockSpec((ROW_BLK, H), lambda i: (i, 0)),
            pl.BlockSpec((1, H), lambda i: (0, 0)),
            pl.BlockSpec((H, H), lambda i: (0, 0)),
            pl.BlockSpec((H, H), lambda i: (0, 0)),
        ],
        out_specs=[
            pl.BlockSpec((ROW_BLK, H), lambda i: (i, 0)),
            pl.BlockSpec((ROW_BLK, H), lambda i: (i, 0)),
            pl.BlockSpec((ROW_BLK, H), lambda i: (i, 0)),
        ],
        out_shape=[
            jax.ShapeDtypeStruct((N, H), jnp.float32),
            jax.ShapeDtypeStruct((N, H), jnp.float32),
            jax.ShapeDtypeStruct((N, H), jnp.float32),
        ],
    )(agg, h_prev, bc_i.reshape(1, H), Wl_i, Wr_i)


def _head_body(agg_ref, hprev_ref, bc_ref, gamma_ref, beta_ref,
               w1_ref, b1_ref, w2_ref, b2_ref, y_ref):
    h = _gelu(agg_ref[...] + bc_ref[...]) + hprev_ref[...]
    mu = jnp.mean(h, axis=-1, keepdims=True)
    var = jnp.mean((h - mu) ** 2, axis=-1, keepdims=True)
    hn = (h - mu) / jnp.sqrt(var + 1e-5) * gamma_ref[...] + beta_ref[...]
    z = _gelu(jnp.dot(hn, w1_ref[...], preferred_element_type=jnp.float32)
              + b1_ref[...])
    y_ref[...] = jnp.sum(z * w2_ref[...], axis=-1, keepdims=True) + b2_ref[...]


def tc_head(agg, h_prev, bc_i, gamma, beta, W1, b1, W2, b2):
    grid = (N // ROW_BLK,)
    return pl.pallas_call(
        _head_body,
        grid=grid,
        in_specs=[
            pl.BlockSpec((ROW_BLK, H), lambda i: (i, 0)),
            pl.BlockSpec((ROW_BLK, H), lambda i: (i, 0)),
            pl.BlockSpec((1, H), lambda i: (0, 0)),
            pl.BlockSpec((1, H), lambda i: (0, 0)),
            pl.BlockSpec((1, H), lambda i: (0, 0)),
            pl.BlockSpec((H, H), lambda i: (0, 0)),
            pl.BlockSpec((1, H), lambda i: (0, 0)),
            pl.BlockSpec((1, H), lambda i: (0, 0)),
            pl.BlockSpec((1, 1), lambda i: (0, 0)),
        ],
        out_specs=pl.BlockSpec((ROW_BLK, 1), lambda i: (i, 0)),
        out_shape=jax.ShapeDtypeStruct((N, 1), jnp.float32),
    )(agg, h_prev, bc_i.reshape(1, H), gamma.reshape(1, H),
      beta.reshape(1, H), W1, b1.reshape(1, H), W2.reshape(1, H),
      b2.reshape(1, 1))


E_SELF = E + N
CHUNK = 128
E_PAD = 172032
A_PER_TILE = E_PAD // 32
NPAD4 = 40960
NSEG = NPAD4 // 16
NPT = 320
ACCR = 328
TRASH_ROW = 324
ASUP = 2048
ACH = 64

NEG_BIG = -1e30

_CP = pltpu.CompilerParams(needs_layout_passes=False)


def _iota16():
    return lax.iota(jnp.int32, 16)


def _fill(ref, start, nvec, value):
    def body(k, _):
        ref[pl.ds(start + 16 * k, 16)] = jnp.full((16,), value, ref.dtype)
        return 0
    lax.fori_loop(0, nvec, body, 0)


def _sc_mesh():
    return plsc.VectorSubcoreMesh(core_axis_name="c", subcore_axis_name="s")


def make_sc_logits():
    mesh = _sc_mesh()

    @functools.partial(
        pl.kernel,
        out_type=jax.ShapeDtypeStruct((E_PAD * HEADS,), jnp.float32),
        mesh=mesh,
        compiler_params=_CP,
        scratch_types=[
            pltpu.VMEM((A_PER_TILE,), jnp.int32),
            pltpu.VMEM((A_PER_TILE,), jnp.int32),
            pltpu.VMEM((CHUNK, H), jnp.float32),
            pltpu.VMEM((CHUNK, H), jnp.float32),
            pltpu.VMEM((A_PER_TILE * HEADS,), jnp.float32),
            pltpu.VMEM((H,), jnp.float32),
            pltpu.SemaphoreType.DMA,
            pltpu.SemaphoreType.DMA,
        ],
    )
    def sc_logits(src_hbm, dst_hbm, xl_hbm, xr_hbm, att_hbm, logit_hbm,
                  srca_v, dsta_v, rows_l, rows_r, lga_v, att_v, sem1, sem2):
        p = lax.axis_index("c")
        s = lax.axis_index("s")
        w = p * 16 + s
        base_w = w * A_PER_TILE

        pltpu.sync_copy(att_hbm, att_v)
        pltpu.sync_copy(src_hbm.at[pl.ds(base_w, A_PER_TILE)], srca_v)
        pltpu.sync_copy(dst_hbm.at[pl.ds(base_w, A_PER_TILE)], dsta_v)

        def clamp_body(k, _):
            v = dsta_v[pl.ds(16 * k, 16)]
            dsta_v[pl.ds(16 * k, 16)] = jnp.minimum(v, N - 1)
            return 0
        lax.fori_loop(0, A_PER_TILE // 16, clamp_body, 0)
        _fill(lga_v, 0, A_PER_TILE * HEADS // 16, 0.0)

        def chunk_body(c, _):
            cb = c * CHUNK
            cl = pltpu.async_copy(
                xl_hbm.at[srca_v.at[pl.ds(cb, CHUNK)]], rows_l, sem1)
            cr = pltpu.async_copy(
                xr_hbm.at[dsta_v.at[pl.ds(cb, CHUNK)]], rows_r, sem2)
            cl.wait()
            cr.wait()

            def edge_body(j, _):
                lbase = (cb + j) * HEADS
                prods = []
                for k in range(16):
                    t = (rows_l[j, pl.ds(16 * k, 16)]
                         + rows_r[j, pl.ds(16 * k, 16)])
                    lr = jnp.maximum(t, 0.2 * t)
                    prods.append(lr * att_v[pl.ds(16 * k, 16)])
                for hh in range(4):
                    p = ((prods[4 * hh] + prods[4 * hh + 1])
                         + (prods[4 * hh + 2] + prods[4 * hh + 3]))
                    idx = jnp.full((16,), lbase + hh, jnp.int32)
                    plsc.addupdate_scatter(lga_v, [idx], p)
                return 0
            lax.fori_loop(0, CHUNK, edge_body, 0)
            return 0
        lax.fori_loop(0, A_PER_TILE // CHUNK, chunk_body, 0)

        pltpu.sync_copy(
            lga_v, logit_hbm.at[pl.ds(base_w * HEADS, A_PER_TILE * HEADS)])

    return sc_logits


def make_sc_stats():
    mesh = _sc_mesh()

    @functools.partial(
        pl.kernel,
        out_type=[jax.ShapeDtypeStruct((2, NPAD4), jnp.float32),
                  jax.ShapeDtypeStruct((2, NPAD4), jnp.float32),
                  jax.ShapeDtypeStruct((32, NPAD4), jnp.float32)],
        mesh=mesh,
        compiler_params=_CP,
        scratch_types=[
            pltpu.VMEM((NPAD4,), jnp.float32),
            pltpu.VMEM((NPAD4,), jnp.float32),
            pltpu.VMEM((A_PER_TILE,), jnp.int32),
            pltpu.VMEM((A_PER_TILE * HEADS,), jnp.float32),
            pltpu.VMEM((NSEG,), jnp.float32),
            pltpu.VMEM((NSEG,), jnp.float32),
            pltpu.VMEM_SHARED((NPAD4,), jnp.float32),
        ],
    )
    def sc_stats(dst_hbm, logit_hbm, m_part, den_part, exch,
                 m_buf, den_buf, dsts_v, lgs_v, acc_v, tmp_v, final_t):
        c = lax.axis_index("c")
        s = lax.axis_index("s")
        w = c * 16 + s
        base_w = w * A_PER_TILE
        iota = _iota16()
        head = iota & 3
        quad = iota >> 2

        pltpu.sync_copy(dst_hbm.at[pl.ds(base_w, A_PER_TILE)], dsts_v)
        pltpu.sync_copy(
            logit_hbm.at[pl.ds(base_w * HEADS, A_PER_TILE * HEADS)], lgs_v)
        _fill(m_buf, 0, NPAD4 // 16, NEG_BIG)
        _fill(den_buf, 0, NPAD4 // 16, 0.0)

        def lane_fidx(v):
            dvec = plsc.load_gather(dsts_v, [4 * v + quad])
            return dvec * 4 + head

        def max_body(v, _):
            fidx = lane_fidx(v)
            val = lgs_v[pl.ds(16 * v, 16)]
            cur = plsc.load_gather(m_buf, [fidx])

            def rbody(t, pn):
                plsc.store_scatter(m_buf, [fidx], val, mask=pn)
                cur2 = plsc.load_gather(m_buf, [fidx])
                return pn & (val > cur2)
            lax.fori_loop(0, 16, rbody, val > cur)
            return 0
        lax.fori_loop(0, A_PER_TILE * HEADS // 16, max_body, 0)

        def reduce_tables(buf_v, out_hbm, is_max):
            plsc.subcore_barrier()
            pltpu.sync_copy(buf_v, exch.at[w])
            plsc.subcore_barrier()
            seg0 = s * NSEG
            pltpu.sync_copy(exch.at[c * 16, pl.ds(seg0, NSEG)], acc_v)

            def tile_body(t, _):
                pltpu.sync_copy(exch.at[c * 16 + t, pl.ds(seg0, NSEG)], tmp_v)

                def vec_body(k, _):
                    a = acc_v[pl.ds(16 * k, 16)]
                    b = tmp_v[pl.ds(16 * k, 16)]
                    acc_v[pl.ds(16 * k, 16)] = (
                        jnp.maximum(a, b) if is_max else a + b)
                    return 0
                lax.fori_loop(0, NSEG // 16, vec_body, 0)
                return 0
            lax.fori_loop(1, 16, tile_body, 0)
            pltpu.sync_copy(acc_v, out_hbm.at[c, pl.ds(seg0, NSEG)])
            pltpu.sync_copy(acc_v, final_t.at[pl.ds(seg0, NSEG)])
            plsc.subcore_barrier()
            pltpu.sync_copy(final_t, buf_v)

        reduce_tables(m_buf, m_part, True)

        def den_body(v, _):
            fidx = lane_fidx(v)
            val = lgs_v[pl.ds(16 * v, 16)]
            mg = plsc.load_gather(m_buf, [fidx])
            a = jnp.exp(val - mg)
            plsc.addupdate_scatter(den_buf, [fidx], a)
            return 0
        lax.fori_loop(0, A_PER_TILE * HEADS // 16, den_body, 0)

        reduce_tables(den_buf, den_part, False)

    return sc_stats


def make_sc_agg():
    mesh = _sc_mesh()

    @functools.partial(
        pl.kernel,
        out_type=jax.ShapeDtypeStruct((32 * NPT, H), jnp.float32),
        mesh=mesh,
        compiler_params=_CP,
        scratch_types=[
            pltpu.VMEM((ACCR, H), jnp.float32),
            pltpu.VMEM((ASUP + 16,), jnp.int32),
            pltpu.VMEM((ASUP + 16,), jnp.int32),
            pltpu.VMEM((ASUP * HEADS + 16,), jnp.float32),
            pltpu.VMEM((ASUP + 64,), jnp.int32),
            pltpu.VMEM((ACH, H), jnp.float32),
            pltpu.VMEM((ACH,), jnp.int32),
            pltpu.VMEM((ACH + 16,), jnp.int32),
            pltpu.VMEM((ACH * HEADS + 16,), jnp.float32),
            pltpu.VMEM((NPT * HEADS,), jnp.float32),
            pltpu.VMEM((NPT * HEADS,), jnp.float32),
            pltpu.VMEM((NPT * HEADS,), jnp.float32),
            pltpu.VMEM((NPT * HEADS,), jnp.float32),
            pltpu.SemaphoreType.DMA,
        ],
    )
    def sc_agg(src_hbm, dst_hbm, logit_hbm, m_part, den_part, xl_hbm,
               agg_hbm, acc, srcc_v, dstc_v, lgc_v, eids_v, rows_v, idx_v,
               drow_v, a_ch, m_loc, rd_loc, tpa, tpb, sem):
        c = lax.axis_index("c")
        s = lax.axis_index("s")
        w = c * 16 + s
        node0 = w * NPT
        iota = _iota16()
        head = iota & 3
        quad = iota >> 2

        t0 = node0 * HEADS
        pltpu.sync_copy(m_part.at[0, pl.ds(t0, NPT * HEADS)], m_loc)
        pltpu.sync_copy(m_part.at[1, pl.ds(t0, NPT * HEADS)], tpa)
        pltpu.sync_copy(den_part.at[0, pl.ds(t0, NPT * HEADS)], rd_loc)
        pltpu.sync_copy(den_part.at[1, pl.ds(t0, NPT * HEADS)], tpb)

        def merge_body(k, _):
            sl = pl.ds(16 * k, 16)
            m0 = m_loc[sl]
            m1 = tpa[sl]
            d0 = rd_loc[sl]
            d1 = tpb[sl]
            mm = jnp.maximum(m0, m1)
            den = d0 * jnp.exp(m0 - mm) + d1 * jnp.exp(m1 - mm)
            m_loc[sl] = mm
            rd_loc[sl] = 1.0 / (den + 1e-16)
            return 0
        lax.fori_loop(0, NPT * HEADS // 16, merge_body, 0)

        def zrow(j, _):
            for k in range(16):
                acc[j, pl.ds(16 * k, 16)] = jnp.zeros((16,), jnp.float32)
            return 0
        lax.fori_loop(0, ACCR, zrow, 0)

        def super_body(sc, _):
            eb = sc * ASUP
            pltpu.sync_copy(src_hbm.at[pl.ds(eb, ASUP)],
                            srcc_v.at[pl.ds(0, ASUP)])
            pltpu.sync_copy(dst_hbm.at[pl.ds(eb, ASUP)],
                            dstc_v.at[pl.ds(0, ASUP)])
            pltpu.sync_copy(
                logit_hbm.at[pl.ds(eb * HEADS, ASUP * HEADS)],
                lgc_v.at[pl.ds(0, ASUP * HEADS)])
            srcc_v[pl.ds(ASUP, 16)] = jnp.zeros((16,), jnp.int32)
            dstc_v[pl.ds(ASUP, 16)] = jnp.full((16,), -1, jnp.int32)
            _fill(eids_v, 0, (ASUP + 64) // 16, ASUP)

            def cmp_body(v, fill):
                dv = dstc_v[pl.ds(16 * v, 16)]
                dloc = dv - node0
                match = (dloc >= 0) & (dloc < NPT)
                ev = iota + 16 * v
                plsc.store_compressed(eids_v.at[pl.ds(fill, 16)], ev,
                                      mask=match)
                pc = plsc.all_reduce_population_count(match)
                return fill + pc[0]
            fill = lax.fori_loop(0, ASUP // 16, cmp_body, 0)
            nch = (fill + ACH - 1) // ACH

            def chunk_body(j, _):
                cb = j * ACH

                def bld_body(v, _):
                    elv = eids_v[pl.ds(cb + 16 * v, 16)]
                    srcs = plsc.load_gather(srcc_v, [elv])
                    idx_v[pl.ds(16 * v, 16)] = srcs
                    dvv = plsc.load_gather(dstc_v, [elv])
                    dloc = dvv - node0
                    drow_v[pl.ds(16 * v, 16)] = jnp.where(
                        dloc >= 0, dloc, TRASH_ROW)
                    return 0
                lax.fori_loop(0, ACH // 16, bld_body, 0)

                def aw_body(v, _):
                    elq = plsc.load_gather(eids_v, [cb + 4 * v + quad])
                    lg = plsc.load_gather(lgc_v, [elq * 4 + head])
                    dvq = plsc.load_gather(dstc_v, [elq])
                    dloc = dvq - node0
                    ok = dloc >= 0
                    fidx = jnp.where(ok, dloc, 0) * 4 + head
                    mg = plsc.load_gather(m_loc, [fidx])
                    rd = plsc.load_gather(rd_loc, [fidx])
                    av = jnp.exp(lg - mg) * rd
                    a_ch[pl.ds(16 * v, 16)] = jnp.where(ok, av, 0.0)
                    return 0
                lax.fori_loop(0, ACH * HEADS // 16, aw_body, 0)

                pltpu.async_copy(xl_hbm.at[idx_v], rows_v, sem).wait()

                def acc_body(j2, _):
                    rv = drow_v[pl.ds(j2, 16)]
                    row = rv[0]
                    av4 = a_ch[pl.ds(HEADS * j2, 16)]
                    aa = (av4[0], av4[1], av4[2], av4[3])
                    for k in range(16):
                        sl = pl.ds(16 * k, 16)
                        acc[row, sl] = (acc[row, sl]
                                        + rows_v[j2, sl] * aa[k // 4])
                    return 0
                lax.fori_loop(0, ACH, acc_body, 0)
                return 0
            lax.fori_loop(0, nch, chunk_body, 0)
            return 0
        lax.fori_loop(0, E_PAD // ASUP, super_body, 0)

        pltpu.sync_copy(acc.at[pl.ds(0, NPT)], agg_hbm.at[pl.ds(node0, NPT)])

    return sc_agg


_SC_LOGITS = None
_SC_STATS = None
_SC_AGG = None


def _sc_kernels():
    global _SC_LOGITS, _SC_STATS, _SC_AGG
    if _SC_LOGITS is None:
        _SC_LOGITS = make_sc_logits()
        _SC_STATS = make_sc_stats()
        _SC_AGG = make_sc_agg()
    return _SC_LOGITS, _SC_STATS, _SC_AGG


def edge_phase(xl, xr, att_i, srcp, dstp):
    sc_logits, sc_stats, sc_agg = _sc_kernels()
    logit = sc_logits(srcp, dstp, xl, xr, att_i.reshape(H))
    m_part, den_part, _ = sc_stats(dstp, logit)
    agg_pad = sc_agg(srcp, dstp, logit, m_part, den_part, xl)
    return agg_pad[:N]


def kernel(x, edge_index, Wp, bp, Wl, Wr, att, bc, gamma, beta,
           W1, b1, W2, b2):
    loop = jnp.arange(N, dtype=jnp.int32)
    npad = E_PAD - E_SELF
    srcp = jnp.concatenate(
        [edge_index[0].astype(jnp.int32), loop,
         jnp.zeros((npad,), jnp.int32)])
    dstp = jnp.concatenate(
        [edge_index[1].astype(jnp.int32), loop,
         jnp.full((npad,), N, jnp.int32)])

    h = tc_proj(x, Wp, bp)
    xl, xr = tc_mm2(h, Wl[0], Wr[0])
    agg = edge_phase(xl, xr, att[0], srcp, dstp)
    for i in range(1, L):
        h, xl, xr = tc_layer(agg, h, bc[i - 1], Wl[i], Wr[i])
        agg = edge_phase(xl, xr, att[i], srcp, dstp)
    y = tc_head(agg, h, bc[L - 1], gamma, beta, W1, b1, W2, b2)
    return y


# speedup vs baseline: 3.4627x; 1.0458x over previous
import functools

import jax
import jax.numpy as jnp
from jax import lax
from jax.experimental import pallas as pl
from jax.experimental.pallas import tpu as pltpu
from jax.experimental.pallas import tpu_sc as plsc

N = 10000
E = 160000
D_IN = 256
H = 256
HEADS = 4
C = H // HEADS
L = 4

ROW_BLK = 400


def _gelu(v):
    return 0.5 * v * (1.0 + lax.erf(v * 0.7071067811865476))


def _proj_body(x_ref, wp_ref, bp_ref, out_ref):
    out_ref[...] = jnp.dot(x_ref[...], wp_ref[...],
                           preferred_element_type=jnp.float32) + bp_ref[...]


def tc_proj(x, Wp, bp):
    grid = (N // ROW_BLK,)
    return pl.pallas_call(
        _proj_body,
        grid=grid,
        in_specs=[
            pl.BlockSpec((ROW_BLK, D_IN), lambda i: (i, 0)),
            pl.BlockSpec((D_IN, H), lambda i: (0, 0)),
            pl.BlockSpec((1, H), lambda i: (0, 0)),
        ],
        out_specs=pl.BlockSpec((ROW_BLK, H), lambda i: (i, 0)),
        out_shape=jax.ShapeDtypeStruct((N, H), jnp.float32),
    )(x, Wp, bp.reshape(1, H))


def _mm2_body(h_ref, wl_ref, wr_ref, xl_ref, xr_ref):
    h = h_ref[...]
    xl_ref[...] = jnp.dot(h, wl_ref[...], preferred_element_type=jnp.float32)
    xr_ref[...] = jnp.dot(h, wr_ref[...], preferred_element_type=jnp.float32)


def tc_mm2(h, Wl_i, Wr_i):
    grid = (N // ROW_BLK,)
    return pl.pallas_call(
        _mm2_body,
        grid=grid,
        in_specs=[
            pl.BlockSpec((ROW_BLK, H), lambda i: (i, 0)),
            pl.BlockSpec((H, H), lambda i: (0, 0)),
            pl.BlockSpec((H, H), lambda i: (0, 0)),
        ],
        out_specs=[
            pl.BlockSpec((ROW_BLK, H), lambda i: (i, 0)),
            pl.BlockSpec((ROW_BLK, H), lambda i: (i, 0)),
        ],
        out_shape=[
            jax.ShapeDtypeStruct((N, H), jnp.float32),
            jax.ShapeDtypeStruct((N, H), jnp.float32),
        ],
    )(h, Wl_i, Wr_i)


def _layer_body(agg_ref, hprev_ref, bc_ref, wl_ref, wr_ref,
                h_ref, xl_ref, xr_ref):
    h = _gelu(agg_ref[...] + bc_ref[...]) + hprev_ref[...]
    h_ref[...] = h
    xl_ref[...] = jnp.dot(h, wl_ref[...], preferred_element_type=jnp.float32)
    xr_ref[...] = jnp.dot(h, wr_ref[...], preferred_element_type=jnp.float32)


def tc_layer(agg, h_prev, bc_i, Wl_i, Wr_i):
    grid = (N // ROW_BLK,)
    return pl.pallas_call(
        _layer_body,
        grid=grid,
        in_specs=[
            pl.BlockSpec((ROW_BLK, H), lambda i: (i, 0)),
            pl.BlockSpec((ROW_BLK, H), lambda i: (i, 0)),
            pl.BlockSpec((1, H), lambda i: (0, 0)),
            pl.BlockSpec((H, H), lambda i: (0, 0)),
            pl.BlockSpec((H, H), lambda i: (0, 0)),
        ],
        out_specs=[
            pl.BlockSpec((ROW_BLK, H), lambda i: (i, 0)),
            pl.BlockSpec((ROW_BLK, H), lambda i: (i, 0)),
            pl.BlockSpec((ROW_BLK, H), lambda i: (i, 0)),
        ],
        out_shape=[
            jax.ShapeDtypeStruct((N, H), jnp.float32),
            jax.ShapeDtypeStruct((N, H), jnp.float32),
            jax.ShapeDtypeStruct((N, H), jnp.float32),
        ],
    )(agg, h_prev, bc_i.reshape(1, H), Wl_i, Wr_i)


def _head_body(agg_ref, hprev_ref, bc_ref, gamma_ref, beta_ref,
               w1_ref, b1_ref, w2_ref, b2_ref, y_ref):
    h = _gelu(agg_ref[...] + bc_ref[...]) + hprev_ref[...]
    mu = jnp.mean(h, axis=-1, keepdims=True)
    var = jnp.mean((h - mu) ** 2, axis=-1, keepdims=True)
    hn = (h - mu) / jnp.sqrt(var + 1e-5) * gamma_ref[...] + beta_ref[...]
    z = _gelu(jnp.dot(hn, w1_ref[...], preferred_element_type=jnp.float32)
              + b1_ref[...])
    y_ref[...] = jnp.sum(z * w2_ref[...], axis=-1, keepdims=True) + b2_ref[...]


def tc_head(agg, h_prev, bc_i, gamma, beta, W1, b1, W2, b2):
    grid = (N // ROW_BLK,)
    return pl.pallas_call(
        _head_body,
        grid=grid,
        in_specs=[
            pl.BlockSpec((ROW_BLK, H), lambda i: (i, 0)),
            pl.BlockSpec((ROW_BLK, H), lambda i: (i, 0)),
            pl.BlockSpec((1, H), lambda i: (0, 0)),
            pl.BlockSpec((1, H), lambda i: (0, 0)),
            pl.BlockSpec((1, H), lambda i: (0, 0)),
            pl.BlockSpec((H, H), lambda i: (0, 0)),
            pl.BlockSpec((1, H), lambda i: (0, 0)),
            pl.BlockSpec((1, H), lambda i: (0, 0)),
            pl.BlockSpec((1, 1), lambda i: (0, 0)),
        ],
        out_specs=pl.BlockSpec((ROW_BLK, 1), lambda i: (i, 0)),
        out_shape=jax.ShapeDtypeStruct((N, 1), jnp.float32),
    )(agg, h_prev, bc_i.reshape(1, H), gamma.reshape(1, H),
      beta.reshape(1, H), W1, b1.reshape(1, H), W2.reshape(1, H),
      b2.reshape(1, 1))


E_SELF = E + N
LCH = 64
E_PAD = 172032
A_PER_TILE = E_PAD // 32
NPAD4 = 40960
NSEG = NPAD4 // 16
NPT = 320
ACCR = 328
TRASH_ROW = 324
ASUP = 2048
ACH = 64

NEG_BIG = -1e30

_CP = pltpu.CompilerParams(needs_layout_passes=False)


def _iota16():
    return lax.iota(jnp.int32, 16)


def _fill(ref, start, nvec, value):
    def body(k, _):
        ref[pl.ds(start + 16 * k, 16)] = jnp.full((16,), value, ref.dtype)
        return 0
    lax.fori_loop(0, nvec, body, 0)


def _sc_mesh():
    return plsc.VectorSubcoreMesh(core_axis_name="c", subcore_axis_name="s")


def make_sc_logits():
    mesh = _sc_mesh()

    @functools.partial(
        pl.kernel,
        out_type=jax.ShapeDtypeStruct((E_PAD * HEADS,), jnp.float32),
        mesh=mesh,
        compiler_params=_CP,
        scratch_types=[
            pltpu.VMEM((A_PER_TILE,), jnp.int32),
            pltpu.VMEM((A_PER_TILE,), jnp.int32),
            pltpu.VMEM((LCH, H), jnp.float32),
            pltpu.VMEM((LCH, H), jnp.float32),
            pltpu.VMEM((LCH, H), jnp.float32),
            pltpu.VMEM((LCH, H), jnp.float32),
            pltpu.VMEM((A_PER_TILE * HEADS,), jnp.float32),
            pltpu.VMEM((H,), jnp.float32),
            pltpu.SemaphoreType.DMA,
            pltpu.SemaphoreType.DMA,
            pltpu.SemaphoreType.DMA,
            pltpu.SemaphoreType.DMA,
        ],
    )
    def sc_logits(src_hbm, dst_hbm, xl_hbm, xr_hbm, att_hbm, logit_hbm,
                  srca_v, dsta_v, rows_la, rows_ra, rows_lb, rows_rb,
                  lga_v, att_v, sal, sar, sbl, sbr):
        p = lax.axis_index("c")
        s = lax.axis_index("s")
        w = p * 16 + s
        base_w = w * A_PER_TILE

        pltpu.sync_copy(att_hbm, att_v)
        pltpu.sync_copy(src_hbm.at[pl.ds(base_w, A_PER_TILE)], srca_v)
        pltpu.sync_copy(dst_hbm.at[pl.ds(base_w, A_PER_TILE)], dsta_v)

        def clamp_body(k, _):
            v = dsta_v[pl.ds(16 * k, 16)]
            dsta_v[pl.ds(16 * k, 16)] = jnp.minimum(v, N - 1)
            return 0
        lax.fori_loop(0, A_PER_TILE // 16, clamp_body, 0)
        _fill(lga_v, 0, A_PER_TILE * HEADS // 16, 0.0)

        def issue(ch, rl, rr, sl, sr):
            cb = ch * LCH
            pltpu.async_copy(xl_hbm.at[srca_v.at[pl.ds(cb, LCH)]], rl, sl)
            pltpu.async_copy(xr_hbm.at[dsta_v.at[pl.ds(cb, LCH)]], rr, sr)

        def drain(rl, rr, sl, sr):
            pltpu.make_async_copy(xl_hbm.at[srca_v.at[pl.ds(0, LCH)]],
                                  rl, sl).wait()
            pltpu.make_async_copy(xr_hbm.at[dsta_v.at[pl.ds(0, LCH)]],
                                  rr, sr).wait()

        def compute(ch, rl, rr):
            cb = ch * LCH

            def edge_body(j, _):
                lbase = (cb + j) * HEADS
                prods = []
                for k in range(16):
                    t = (rl[j, pl.ds(16 * k, 16)]
                         + rr[j, pl.ds(16 * k, 16)])
                    lr = jnp.maximum(t, 0.2 * t)
                    prods.append(lr * att_v[pl.ds(16 * k, 16)])
                for hh in range(4):
                    p = ((prods[4 * hh] + prods[4 * hh + 1])
                         + (prods[4 * hh + 2] + prods[4 * hh + 3]))
                    idx = jnp.full((16,), lbase + hh, jnp.int32)
                    plsc.addupdate_scatter(lga_v, [idx], p)
                return 0
            lax.fori_loop(0, LCH, edge_body, 0)

        npairs = A_PER_TILE // (2 * LCH)
        issue(0, rows_la, rows_ra, sal, sar)

        def pair_body(c2, _):
            drain(rows_la, rows_ra, sal, sar)
            issue(2 * c2 + 1, rows_lb, rows_rb, sbl, sbr)
            compute(2 * c2, rows_la, rows_ra)
            drain(rows_lb, rows_rb, sbl, sbr)

            @pl.when(c2 < npairs - 1)
            def _():
                issue(2 * c2 + 2, rows_la, rows_ra, sal, sar)
            compute(2 * c2 + 1, rows_lb, rows_rb)
            return 0
        lax.fori_loop(0, npairs, pair_body, 0)

        pltpu.sync_copy(
            lga_v, logit_hbm.at[pl.ds(base_w * HEADS, A_PER_TILE * HEADS)])

    return sc_logits


def make_sc_stats():
    mesh = _sc_mesh()

    @functools.partial(
        pl.kernel,
        out_type=[jax.ShapeDtypeStruct((2, NPAD4), jnp.float32),
                  jax.ShapeDtypeStruct((2, NPAD4), jnp.float32),
                  jax.ShapeDtypeStruct((32, NPAD4), jnp.float32)],
        mesh=mesh,
        compiler_params=_CP,
        scratch_types=[
            pltpu.VMEM((NPAD4,), jnp.float32),
            pltpu.VMEM((NPAD4,), jnp.float32),
            pltpu.VMEM((A_PER_TILE,), jnp.int32),
            pltpu.VMEM((A_PER_TILE * HEADS,), jnp.float32),
            pltpu.VMEM((NSEG,), jnp.float32),
            pltpu.VMEM((NSEG,), jnp.float32),
            pltpu.VMEM_SHARED((NPAD4,), jnp.float32),
        ],
    )
    def sc_stats(dst_hbm, logit_hbm, m_part, den_part, exch,
                 m_buf, den_buf, dsts_v, lgs_v, acc_v, tmp_v, final_t):
        c = lax.axis_index("c")
        s = lax.axis_index("s")
        w = c * 16 + s
        base_w = w * A_PER_TILE
        iota = _iota16()
        head = iota & 3
        quad = iota >> 2

        pltpu.sync_copy(dst_hbm.at[pl.ds(base_w, A_PER_TILE)], dsts_v)
        pltpu.sync_copy(
            logit_hbm.at[pl.ds(base_w * HEADS, A_PER_TILE * HEADS)], lgs_v)
        _fill(m_buf, 0, NPAD4 // 16, NEG_BIG)
        _fill(den_buf, 0, NPAD4 // 16, 0.0)

        def lane_fidx(v):
            dvec = plsc.load_gather(dsts_v, [4 * v + quad])
            return dvec * 4 + head

        def max_body(v, _):
            fidx = lane_fidx(v)
            val = lgs_v[pl.ds(16 * v, 16)]
            cur = plsc.load_gather(m_buf, [fidx])

            def rbody(t, pn):
                plsc.store_scatter(m_buf, [fidx], val, mask=pn)
                cur2 = plsc.load_gather(m_buf, [fidx])
                return pn & (val > cur2)
            lax.fori_loop(0, 16, rbody, val > cur)
            return 0
        lax.fori_loop(0, A_PER_TILE * HEADS // 16, max_body, 0)

        def reduce_tables(buf_v, out_hbm, is_max):
            plsc.subcore_barrier()
            pltpu.sync_copy(buf_v, exch.at[w])
            plsc.subcore_barrier()
            seg0 = s * NSEG
            pltpu.sync_copy(exch.at[c * 16, pl.ds(seg0, NSEG)], acc_v)

            def tile_body(t, _):
                pltpu.sync_copy(exch.at[c * 16 + t, pl.ds(seg0, NSEG)], tmp_v)

                def vec_body(k, _):
                    a = acc_v[pl.ds(16 * k, 16)]
                    b = tmp_v[pl.ds(16 * k, 16)]
                    acc_v[pl.ds(16 * k, 16)] = (
                        jnp.maximum(a, b) if is_max else a + b)
                    return 0
                lax.fori_loop(0, NSEG // 16, vec_body, 0)
                return 0
            lax.fori_loop(1, 16, tile_body, 0)
            pltpu.sync_copy(acc_v, out_hbm.at[c, pl.ds(seg0, NSEG)])
            pltpu.sync_copy(acc_v, final_t.at[pl.ds(seg0, NSEG)])
            plsc.subcore_barrier()
            pltpu.sync_copy(final_t, buf_v)

        reduce_tables(m_buf, m_part, True)

        def den_body(v, _):
            fidx = lane_fidx(v)
            val = lgs_v[pl.ds(16 * v, 16)]
            mg = plsc.load_gather(m_buf, [fidx])
            a = jnp.exp(val - mg)
            plsc.addupdate_scatter(den_buf, [fidx], a)
            return 0
        lax.fori_loop(0, A_PER_TILE * HEADS // 16, den_body, 0)

        reduce_tables(den_buf, den_part, False)

    return sc_stats


def make_sc_agg():
    mesh = _sc_mesh()

    @functools.partial(
        pl.kernel,
        out_type=jax.ShapeDtypeStruct((32 * NPT, H), jnp.float32),
        mesh=mesh,
        compiler_params=_CP,
        scratch_types=[
            pltpu.VMEM((ACCR, H), jnp.float32),
            pltpu.VMEM((ASUP + 16,), jnp.int32),
            pltpu.VMEM((ASUP + 16,), jnp.int32),
            pltpu.VMEM((ASUP * HEADS + 16,), jnp.float32),
            pltpu.VMEM((ASUP + 64,), jnp.int32),
            pltpu.VMEM((ACH, H), jnp.float32),
            pltpu.VMEM((ACH,), jnp.int32),
            pltpu.VMEM((ACH + 16,), jnp.int32),
            pltpu.VMEM((ACH * HEADS + 16,), jnp.float32),
            pltpu.VMEM((NPT * HEADS,), jnp.float32),
            pltpu.VMEM((NPT * HEADS,), jnp.float32),
            pltpu.VMEM((NPT * HEADS,), jnp.float32),
            pltpu.VMEM((NPT * HEADS,), jnp.float32),
            pltpu.SemaphoreType.DMA,
        ],
    )
    def sc_agg(src_hbm, dst_hbm, logit_hbm, m_part, den_part, xl_hbm,
               agg_hbm, acc, srcc_v, dstc_v, lgc_v, eids_v, rows_v, idx_v,
               drow_v, a_ch, m_loc, rd_loc, tpa, tpb, sem):
        c = lax.axis_index("c")
        s = lax.axis_index("s")
        w = c * 16 + s
        node0 = w * NPT
        iota = _iota16()
        head = iota & 3
        quad = iota >> 2

        t0 = node0 * HEADS
        pltpu.sync_copy(m_part.at[0, pl.ds(t0, NPT * HEADS)], m_loc)
        pltpu.sync_copy(m_part.at[1, pl.ds(t0, NPT * HEADS)], tpa)
        pltpu.sync_copy(den_part.at[0, pl.ds(t0, NPT * HEADS)], rd_loc)
        pltpu.sync_copy(den_part.at[1, pl.ds(t0, NPT * HEADS)], tpb)

        def merge_body(k, _):
            sl = pl.ds(16 * k, 16)
            m0 = m_loc[sl]
            m1 = tpa[sl]
            d0 = rd_loc[sl]
            d1 = tpb[sl]
            mm = jnp.maximum(m0, m1)
            den = d0 * jnp.exp(m0 - mm) + d1 * jnp.exp(m1 - mm)
            m_loc[sl] = mm
            rd_loc[sl] = 1.0 / (den + 1e-16)
            return 0
        lax.fori_loop(0, NPT * HEADS // 16, merge_body, 0)

        def zrow(j, _):
            for k in range(16):
                acc[j, pl.ds(16 * k, 16)] = jnp.zeros((16,), jnp.float32)
            return 0
        lax.fori_loop(0, ACCR, zrow, 0)

        def super_body(sc, _):
            eb = sc * ASUP
            pltpu.sync_copy(src_hbm.at[pl.ds(eb, ASUP)],
                            srcc_v.at[pl.ds(0, ASUP)])
            pltpu.sync_copy(dst_hbm.at[pl.ds(eb, ASUP)],
                            dstc_v.at[pl.ds(0, ASUP)])
            pltpu.sync_copy(
                logit_hbm.at[pl.ds(eb * HEADS, ASUP * HEADS)],
                lgc_v.at[pl.ds(0, ASUP * HEADS)])
            srcc_v[pl.ds(ASUP, 16)] = jnp.zeros((16,), jnp.int32)
            dstc_v[pl.ds(ASUP, 16)] = jnp.full((16,), -1, jnp.int32)
            _fill(eids_v, 0, (ASUP + 64) // 16, ASUP)

            def cmp_body(v, fill):
                dv = dstc_v[pl.ds(16 * v, 16)]
                dloc = dv - node0
                match = (dloc >= 0) & (dloc < NPT)
                ev = iota + 16 * v
                plsc.store_compressed(eids_v.at[pl.ds(fill, 16)], ev,
                                      mask=match)
                pc = plsc.all_reduce_population_count(match)
                return fill + pc[0]
            fill = lax.fori_loop(0, ASUP // 16, cmp_body, 0)
            nch = (fill + ACH - 1) // ACH

            def chunk_body(j, _):
                cb = j * ACH

                def bld_body(v, _):
                    elv = eids_v[pl.ds(cb + 16 * v, 16)]
                    srcs = plsc.load_gather(srcc_v, [elv])
                    idx_v[pl.ds(16 * v, 16)] = srcs
                    dvv = plsc.load_gather(dstc_v, [elv])
                    dloc = dvv - node0
                    drow_v[pl.ds(16 * v, 16)] = jnp.where(
                        dloc >= 0, dloc, TRASH_ROW)
                    return 0
                lax.fori_loop(0, ACH // 16, bld_body, 0)

                def aw_body(v, _):
                    elq = plsc.load_gather(eids_v, [cb + 4 * v + quad])
                    lg = plsc.load_gather(lgc_v, [elq * 4 + head])
                    dvq = plsc.load_gather(dstc_v, [elq])
                    dloc = dvq - node0
                    ok = dloc >= 0
                    fidx = jnp.where(ok, dloc, 0) * 4 + head
                    mg = plsc.load_gather(m_loc, [fidx])
                    rd = plsc.load_gather(rd_loc, [fidx])
                    av = jnp.exp(lg - mg) * rd
                    a_ch[pl.ds(16 * v, 16)] = jnp.where(ok, av, 0.0)
                    return 0
                lax.fori_loop(0, ACH * HEADS // 16, aw_body, 0)

                pltpu.async_copy(xl_hbm.at[idx_v], rows_v, sem).wait()

                def acc_body(j2, _):
                    rv = drow_v[pl.ds(j2, 16)]
                    row = rv[0]
                    av4 = a_ch[pl.ds(HEADS * j2, 16)]
                    aa = (av4[0], av4[1], av4[2], av4[3])
                    for k in range(16):
                        sl = pl.ds(16 * k, 16)
                        acc[row, sl] = (acc[row, sl]
                                        + rows_v[j2, sl] * aa[k // 4])
                    return 0
                lax.fori_loop(0, ACH, acc_body, 0)
                return 0
            lax.fori_loop(0, nch, chunk_body, 0)
            return 0
        lax.fori_loop(0, E_PAD // ASUP, super_body, 0)

        pltpu.sync_copy(acc.at[pl.ds(0, NPT)], agg_hbm.at[pl.ds(node0, NPT)])

    return sc_agg


_SC_LOGITS = None
_SC_STATS = None
_SC_AGG = None


def _sc_kernels():
    global _SC_LOGITS, _SC_STATS, _SC_AGG
    if _SC_LOGITS is None:
        _SC_LOGITS = make_sc_logits()
        _SC_STATS = make_sc_stats()
        _SC_AGG = make_sc_agg()
    return _SC_LOGITS, _SC_STATS, _SC_AGG


def edge_phase(xl, xr, att_i, srcp, dstp):
    sc_logits, sc_stats, sc_agg = _sc_kernels()
    logit = sc_logits(srcp, dstp, xl, xr, att_i.reshape(H))
    m_part, den_part, _ = sc_stats(dstp, logit)
    agg_pad = sc_agg(srcp, dstp, logit, m_part, den_part, xl)
    return agg_pad[:N]


def kernel(x, edge_index, Wp, bp, Wl, Wr, att, bc, gamma, beta,
           W1, b1, W2, b2):
    loop = jnp.arange(N, dtype=jnp.int32)
    npad = E_PAD - E_SELF
    srcp = jnp.concatenate(
        [edge_index[0].astype(jnp.int32), loop,
         jnp.zeros((npad,), jnp.int32)])
    dstp = jnp.concatenate(
        [edge_index[1].astype(jnp.int32), loop,
         jnp.full((npad,), N, jnp.int32)])

    h = tc_proj(x, Wp, bp)
    xl, xr = tc_mm2(h, Wl[0], Wr[0])
    agg = edge_phase(xl, xr, att[0], srcp, dstp)
    for i in range(1, L):
        h, xl, xr = tc_layer(agg, h, bc[i - 1], Wl[i], Wr[i])
        agg = edge_phase(xl, xr, att[i], srcp, dstp)
    y = tc_head(agg, h, bc[L - 1], gamma, beta, W1, b1, W2, b2)
    return y
